# baseline (speedup 1.0000x reference)
import jax
import jax.numpy as jnp
from jax import lax
from jax.experimental import pallas as pl
from jax.experimental.pallas import tpu as pltpu
from jax.experimental.pallas import tpu_sc as plsc

_ROWS = 100000
_NW = 32
_CHUNK = 200
_NCH = _ROWS // _CHUNK
_ITERS = -(-_NCH // _NW)
_DEPTH = 4


def _cp(src, dst, cid, sems, i):
    return pltpu.make_async_copy(
        src.at[pl.ds(cid * _CHUNK, _CHUNK), :],
        dst.at[pl.ds(cid * _CHUNK, _CHUNK), :],
        sems.at[i % _DEPTH])


def _pipe(src, dst, wid, sems):
    for i in range(_ITERS + _DEPTH):
        kd = i - _DEPTH
        if 0 <= kd < _ITERS:
            cd = wid + kd * _NW

            @pl.when(cd < _NCH)
            def _(cd=cd, kd=kd):
                _cp(src, dst, cd, sems, kd).wait()
        if i < _ITERS:
            ci = wid + i * _NW

            @pl.when(ci < _NCH)
            def _(ci=ci, i=i):
                _cp(src, dst, ci, sems, i).start()


def _sc_body(c_in, n_in, u_in, c_out, n_out, u_out, sems):
    wid = lax.axis_index("s") * 2 + lax.axis_index("c")
    _pipe(c_in, c_out, wid, sems)
    _pipe(n_in, n_out, wid, sems)
    _pipe(u_in, u_out, wid, sems)


def kernel(c_embeddings, n_embeddings, u_embeddings):
    mesh = plsc.VectorSubcoreMesh(
        core_axis_name="c", subcore_axis_name="s", num_cores=2, num_subcores=16)
    run = pl.kernel(
        _sc_body,
        out_type=(
            jax.ShapeDtypeStruct(c_embeddings.shape, c_embeddings.dtype),
            jax.ShapeDtypeStruct(n_embeddings.shape, n_embeddings.dtype),
            jax.ShapeDtypeStruct(u_embeddings.shape, u_embeddings.dtype),
        ),
        mesh=mesh,
        scratch_types=[
            pltpu.SemaphoreType.DMA((_DEPTH,)),
        ],
    )
    out = run(c_embeddings, n_embeddings, u_embeddings)
    return (out[0], out[1], out[2])


# speedup vs baseline: 26.7600x; 26.7600x over previous
import jax
import jax.numpy as jnp
from jax import lax
from jax.experimental import pallas as pl
from jax.experimental.pallas import tpu as pltpu
from jax.experimental.pallas import tpu_sc as plsc

_ROWS = 100000


_TC_CHUNK = 2000
_TC_NCH = _ROWS // _TC_CHUNK
_TC_NBUF = 12
_TC_LEAD = 6


def _tc_in(src, bufs, sems, i):
    b = i % _TC_NBUF
    return pltpu.make_async_copy(
        src.at[pl.ds(i * _TC_CHUNK, _TC_CHUNK), :], bufs.at[b], sems.at[b])


def _tc_out(dst, bufs, sems, k):
    b = k % _TC_NBUF
    return pltpu.make_async_copy(
        bufs.at[b], dst.at[pl.ds(k * _TC_CHUNK, _TC_CHUNK), :], sems.at[b])


def _tc_pipe(src, dst, bufs, in_sems, out_sems):
    for i in range(_TC_NCH + _TC_LEAD):
        if i < _TC_NCH:
            if i >= _TC_NBUF:
                _tc_out(dst, bufs, out_sems, i - _TC_NBUF).wait()
            _tc_in(src, bufs, in_sems, i).start()
        k = i - _TC_LEAD
        if k >= 0:
            _tc_in(src, bufs, in_sems, k).wait()
            _tc_out(dst, bufs, out_sems, k).start()
    for k in range(max(0, _TC_NCH - _TC_NBUF), _TC_NCH):
        _tc_out(dst, bufs, out_sems, k).wait()


def _tc_body(c_in, u_in, c_out, u_out, buf128, buf64, in_sems, out_sems):
    _tc_pipe(c_in, c_out, buf128, in_sems, out_sems)
    _tc_pipe(u_in, u_out, buf64, in_sems, out_sems)


def _tc_copy(c, u):
    return pl.pallas_call(
        _tc_body,
        in_specs=[pl.BlockSpec(memory_space=pl.ANY)] * 2,
        out_specs=[pl.BlockSpec(memory_space=pl.ANY)] * 2,
        out_shape=(
            jax.ShapeDtypeStruct(c.shape, c.dtype),
            jax.ShapeDtypeStruct(u.shape, u.dtype),
        ),
        scratch_shapes=[
            pltpu.MemorySpace.VMEM((_TC_NBUF, _TC_CHUNK, 128), jnp.float32),
            pltpu.MemorySpace.VMEM((_TC_NBUF, _TC_CHUNK, 64), jnp.float32),
            pltpu.SemaphoreType.DMA((_TC_NBUF,)),
            pltpu.SemaphoreType.DMA((_TC_NBUF,)),
        ],
    )(c, u)


_NW = 32
_SC_CHUNK = 200
_SC_NCH = _ROWS // _SC_CHUNK
_SC_ITERS = -(-_SC_NCH // _NW)
_SC_NBUF = 2


def _sc_in(src, cid, bufs, sems, i):
    b = i % _SC_NBUF
    return pltpu.make_async_copy(
        src.at[pl.ds(cid * _SC_CHUNK, _SC_CHUNK), :], bufs.at[b], sems.at[b])


def _sc_out(dst, cid, bufs, sems, k):
    b = k % _SC_NBUF
    return pltpu.make_async_copy(
        bufs.at[b], dst.at[pl.ds(cid * _SC_CHUNK, _SC_CHUNK), :], sems.at[b])


def _sc_body(n_in, n_out, bufs, in_sems, out_sems):
    wid = lax.axis_index("s") * 2 + lax.axis_index("c")
    for i in range(_SC_ITERS + _SC_NBUF):
        kd = i - _SC_NBUF
        if 0 <= kd < _SC_ITERS:
            cd = wid + kd * _NW

            @pl.when(cd < _SC_NCH)
            def _(cd=cd, kd=kd):
                _sc_out(n_out, cd, bufs, out_sems, kd).wait()
        if i < _SC_ITERS:
            ci = wid + i * _NW

            @pl.when(ci < _SC_NCH)
            def _(ci=ci, i=i):
                _sc_in(n_in, ci, bufs, in_sems, i).start()
        k = i - 1
        if 0 <= k < _SC_ITERS:
            ck = wid + k * _NW

            @pl.when(ck < _SC_NCH)
            def _(ck=ck, k=k):
                _sc_in(n_in, ck, bufs, in_sems, k).wait()
                _sc_out(n_out, ck, bufs, out_sems, k).start()


def _sc_copy(n):
    mesh = plsc.VectorSubcoreMesh(
        core_axis_name="c", subcore_axis_name="s", num_cores=2, num_subcores=16)
    run = pl.kernel(
        _sc_body,
        out_type=jax.ShapeDtypeStruct(n.shape, n.dtype),
        mesh=mesh,
        scratch_types=[
            pltpu.VMEM((_SC_NBUF, _SC_CHUNK, 128), jnp.float32),
            pltpu.SemaphoreType.DMA((_SC_NBUF,)),
            pltpu.SemaphoreType.DMA((_SC_NBUF,)),
        ],
    )
    return run(n)


def kernel(c_embeddings, n_embeddings, u_embeddings):
    n_out = _sc_copy(n_embeddings)
    c_out, u_out = _tc_copy(c_embeddings, u_embeddings)
    return (c_out, n_out, u_out)
